# baseline (speedup 1.0000x reference)
import functools

import jax
import jax.numpy as jnp
from jax import lax
from jax.experimental import pallas as pl
from jax.experimental.pallas import tpu as pltpu
from jax.experimental.pallas import tpu_sc as plsc

L = 16
NC = 2
NS = 16
NW = NC * NS
P = 128


def _build_sc_interp(B, C, H, W, N):
    pts_total = B * N
    assert pts_total % NW == 0
    ppw = pts_total // NW
    assert ppw % (2 * P) == 0
    n_chunks = ppw // P
    half = n_chunks // 2
    assert N % ppw == 0
    wpb = N // ppw

    mesh = plsc.VectorSubcoreMesh(core_axis_name="c", subcore_axis_name="s",
                                  num_cores=NC, num_subcores=NS)

    @functools.partial(
        pl.kernel,
        out_type=jax.ShapeDtypeStruct((B * C * N,), jnp.float32),
        mesh=mesh,
        scratch_types=[
            pltpu.VMEM((ppw,), jnp.float32),
            pltpu.VMEM((ppw,), jnp.float32),
            pltpu.VMEM((P,), jnp.float32),
            pltpu.VMEM((P,), jnp.float32),
            pltpu.VMEM((P,), jnp.float32),
            pltpu.VMEM((P,), jnp.float32),
            [[[pltpu.VMEM((P,), jnp.int32) for _ in range(4)]
              for _ in range(2)] for _ in range(2)],
            [[[pltpu.VMEM((P,), jnp.int32) for _ in range(4)]
              for _ in range(2)] for _ in range(2)],
            pltpu.VMEM((4 * ppw,), jnp.float32),
            pltpu.SemaphoreType.DMA,
            pltpu.SemaphoreType.DMA,
        ],
    )
    def sc_interp(table_hbm, xy_hbm, out_hbm,
                  xs_v, ys_v, wxa, wya, wxb, wyb,
                  idx_v, g_v, out_v, semA, semB):
        cid = lax.axis_index("c")
        sid = lax.axis_index("s")
        wid = sid * NC + cid
        b = wid // wpb
        n_base = (wid % wpb) * ppw
        HW = H * W
        sems = (semA, semB)
        wxs = (wxa, wxb)
        wys = (wya, wyb)

        pltpu.sync_copy(xy_hbm.at[pl.ds(b * 2 * N + n_base, ppw)], xs_v)
        pltpu.sync_copy(xy_hbm.at[pl.ds(b * 2 * N + N + n_base, ppw)], ys_v)

        def phase1_all(buf, chunk):
            co = chunk * P
            for g in range(P // L):
                sl = pl.ds(g * L, L)
                s2 = pl.ds(co + g * L, L)
                x = xs_v[s2] * float(W - 1)
                y = ys_v[s2] * float(H - 1)
                x0 = jnp.clip(x.astype(jnp.int32), 0, W - 2)
                y0 = jnp.clip(y.astype(jnp.int32), 0, H - 2)
                wxs[buf][sl] = x - x0.astype(jnp.float32)
                wys[buf][sl] = y - y0.astype(jnp.float32)
                base = (b * C * H + y0) * W + x0
                for c in range(C):
                    fc = base + c * HW
                    idx_v[buf][0][c][sl] = fc
                    idx_v[buf][1][c][sl] = fc + 1

        def fire(buf):
            for k in range(2):
                for c in range(C):
                    pltpu.async_copy(table_hbm.at[idx_v[buf][k][c]],
                                     g_v[buf][k][c], sems[buf])

        def drain(buf):
            for k in range(2):
                for c in range(C):
                    pltpu.make_async_copy(table_hbm.at[idx_v[buf][k][c]],
                                          g_v[buf][k][c], sems[buf]).wait()

        def phase3(buf, chunk):
            co = chunk * P
            for g in range(P // L):
                sl = pl.ds(g * L, L)
                wx = wxs[buf][sl]
                wy = wys[buf][sl]
                ex = 1.0 - wx
                ey = 1.0 - wy
                gb = g_v[buf]
                for c in range(C):
                    p0 = gb[0][c][sl]
                    p1 = gb[1][c][sl]
                    v00 = lax.bitcast_convert_type(p0 << 16, jnp.float32)
                    v10 = lax.bitcast_convert_type(p0 & -65536, jnp.float32)
                    v01 = lax.bitcast_convert_type(p1 << 16, jnp.float32)
                    v11 = lax.bitcast_convert_type(p1 & -65536, jnp.float32)
                    out_v[pl.ds(c * ppw + co + g * L, L)] = (
                        (v00 * ex + v01 * wx) * ey
                        + (v10 * ex + v11 * wx) * wy)

        phase1_all(0, 0)
        fire(0)

        def pair_body(i, carry):
            c0 = 2 * i
            phase1_all(1, c0 + 1)
            fire(1)
            drain(0)
            phase3(0, c0)
            phase1_all(0, c0 + 2)
            fire(0)
            drain(1)
            phase3(1, c0 + 1)
            return carry

        lax.fori_loop(0, half - 1, pair_body, 0)

        phase1_all(1, n_chunks - 1)
        fire(1)
        drain(0)
        phase3(0, n_chunks - 2)
        drain(1)
        phase3(1, n_chunks - 1)

        for c in range(C):
            pltpu.sync_copy(
                out_v.at[pl.ds(c * ppw, ppw)],
                out_hbm.at[pl.ds((b * C + c) * N + n_base, ppw)])

    return sc_interp


def _pack_half(Rh, W):
    n = Rh.size
    Rf = Rh.reshape(n)
    lo = lax.bitcast_convert_type(
        Rf.astype(jnp.bfloat16), jnp.uint16).astype(jnp.uint32)
    Rf_dn = jnp.concatenate([Rf[W:], Rf[-W:]])
    hi = lax.bitcast_convert_type(
        Rf_dn.astype(jnp.bfloat16), jnp.uint16).astype(jnp.uint32)
    return lax.bitcast_convert_type(lo | (hi << 16), jnp.int32)


@jax.jit
def kernel(R, XY_pc):
    B, C, H, W = R.shape
    N = XY_pc.shape[-1]
    nsplit = 4
    hb = B // nsplit
    sc_interp = _build_sc_interp(hb, C, H, W, N)
    outs = []
    for i in range(nsplit):
        Rh = R[i * hb:(i + 1) * hb]
        table = _pack_half(Rh, W)
        xy = XY_pc[i * hb:(i + 1) * hb].reshape(hb * 2 * N)
        outs.append(sc_interp(table, xy).reshape(hb, C, N))
    return jnp.concatenate(outs, axis=0)


# speedup vs baseline: 1.0139x; 1.0139x over previous
import functools

import jax
import jax.numpy as jnp
from jax import lax
from jax.experimental import pallas as pl
from jax.experimental.pallas import tpu as pltpu
from jax.experimental.pallas import tpu_sc as plsc

L = 16
NC = 2
NS = 16
NW = NC * NS
P = 128


def _build_sc_interp(B, C, H, W, N):
    pts_total = B * N
    assert pts_total % NW == 0
    ppw = pts_total // NW
    assert ppw % (2 * P) == 0
    n_chunks = ppw // P
    half = n_chunks // 2
    assert N % ppw == 0
    wpb = N // ppw

    mesh = plsc.VectorSubcoreMesh(core_axis_name="c", subcore_axis_name="s",
                                  num_cores=NC, num_subcores=NS)

    @functools.partial(
        pl.kernel,
        out_type=jax.ShapeDtypeStruct((B * C * N,), jnp.float32),
        mesh=mesh,
        scratch_types=[
            pltpu.VMEM((ppw,), jnp.float32),
            pltpu.VMEM((ppw,), jnp.float32),
            pltpu.VMEM((P,), jnp.float32),
            pltpu.VMEM((P,), jnp.float32),
            pltpu.VMEM((P,), jnp.float32),
            pltpu.VMEM((P,), jnp.float32),
            [[[pltpu.VMEM((P,), jnp.int32) for _ in range(4)]
              for _ in range(2)] for _ in range(2)],
            [[[pltpu.VMEM((P,), jnp.int32) for _ in range(4)]
              for _ in range(2)] for _ in range(2)],
            pltpu.VMEM((4 * ppw,), jnp.float32),
            pltpu.SemaphoreType.DMA,
            pltpu.SemaphoreType.DMA,
        ],
    )
    def sc_interp(table_hbm, xy_hbm, out_hbm,
                  xs_v, ys_v, wxa, wya, wxb, wyb,
                  idx_v, g_v, out_v, semA, semB):
        cid = lax.axis_index("c")
        sid = lax.axis_index("s")
        wid = sid * NC + cid
        b = wid // wpb
        n_base = (wid % wpb) * ppw
        HW = H * W
        sems = (semA, semB)
        wxs = (wxa, wxb)
        wys = (wya, wyb)

        pltpu.sync_copy(xy_hbm.at[pl.ds(b * 2 * N + n_base, ppw)], xs_v)
        pltpu.sync_copy(xy_hbm.at[pl.ds(b * 2 * N + N + n_base, ppw)], ys_v)

        def phase1_all(buf, chunk):
            co = chunk * P
            for g in range(P // L):
                sl = pl.ds(g * L, L)
                s2 = pl.ds(co + g * L, L)
                x = xs_v[s2] * float(W - 1)
                y = ys_v[s2] * float(H - 1)
                x0 = jnp.clip(x.astype(jnp.int32), 0, W - 2)
                y0 = jnp.clip(y.astype(jnp.int32), 0, H - 2)
                wxs[buf][sl] = x - x0.astype(jnp.float32)
                wys[buf][sl] = y - y0.astype(jnp.float32)
                base = (b * C * H + y0) * W + x0
                for c in range(C):
                    fc = base + c * HW
                    idx_v[buf][0][c][sl] = fc
                    idx_v[buf][1][c][sl] = fc + 1

        def fire(buf):
            for k in range(2):
                for c in range(C):
                    pltpu.async_copy(table_hbm.at[idx_v[buf][k][c]],
                                     g_v[buf][k][c], sems[buf])

        def drain(buf):
            for k in range(2):
                for c in range(C):
                    pltpu.make_async_copy(table_hbm.at[idx_v[buf][k][c]],
                                          g_v[buf][k][c], sems[buf]).wait()

        def phase3(buf, chunk):
            co = chunk * P
            for g in range(P // L):
                sl = pl.ds(g * L, L)
                wx = wxs[buf][sl]
                wy = wys[buf][sl]
                ex = 1.0 - wx
                ey = 1.0 - wy
                gb = g_v[buf]
                for c in range(C):
                    p0 = gb[0][c][sl]
                    p1 = gb[1][c][sl]
                    v00 = lax.bitcast_convert_type(p0 << 16, jnp.float32)
                    v10 = lax.bitcast_convert_type(p0 & -65536, jnp.float32)
                    v01 = lax.bitcast_convert_type(p1 << 16, jnp.float32)
                    v11 = lax.bitcast_convert_type(p1 & -65536, jnp.float32)
                    out_v[pl.ds(c * ppw + co + g * L, L)] = (
                        (v00 * ex + v01 * wx) * ey
                        + (v10 * ex + v11 * wx) * wy)

        phase1_all(0, 0)
        fire(0)

        def pair_body(i, carry):
            c0 = 2 * i
            phase1_all(1, c0 + 1)
            fire(1)
            drain(0)
            phase3(0, c0)
            phase1_all(0, c0 + 2)
            fire(0)
            drain(1)
            phase3(1, c0 + 1)
            return carry

        lax.fori_loop(0, half - 1, pair_body, 0)

        phase1_all(1, n_chunks - 1)
        fire(1)
        drain(0)
        phase3(0, n_chunks - 2)
        drain(1)
        phase3(1, n_chunks - 1)

        for c in range(C):
            pltpu.sync_copy(
                out_v.at[pl.ds(c * ppw, ppw)],
                out_hbm.at[pl.ds((b * C + c) * N + n_base, ppw)])

    return sc_interp


def _pack_half(Rh, W):
    n = Rh.size
    Rf = Rh.reshape(n)
    lo = lax.bitcast_convert_type(
        Rf.astype(jnp.bfloat16), jnp.uint16).astype(jnp.uint32)
    Rf_dn = jnp.concatenate([Rf[W:], Rf[-W:]])
    hi = lax.bitcast_convert_type(
        Rf_dn.astype(jnp.bfloat16), jnp.uint16).astype(jnp.uint32)
    return lax.bitcast_convert_type(lo | (hi << 16), jnp.int32)


@jax.jit
def kernel(R, XY_pc):
    B, C, H, W = R.shape
    N = XY_pc.shape[-1]
    nsplit = 2
    hb = B // nsplit
    sc_interp = _build_sc_interp(hb, C, H, W, N)
    outs = []
    for i in range(nsplit):
        Rh = R[i * hb:(i + 1) * hb]
        table = _pack_half(Rh, W)
        xy = XY_pc[i * hb:(i + 1) * hb].reshape(hb * 2 * N)
        outs.append(sc_interp(table, xy).reshape(hb, C, N))
    return jnp.concatenate(outs, axis=0)
